# Initial kernel scaffold; baseline (speedup 1.0000x reference)
import jax
import jax.numpy as jnp
from jax.experimental import pallas as pl


def kernel(x, A, W_pool0, b_pool0, W_self0, b_self0, W_neigh0, b_neigh0, W_pool1, b_pool1, W_self1, b_self1, W_neigh1, b_neigh1):
    raise NotImplementedError("write your pallas kernel here")


# speedup vs baseline: 9.0851x; 9.0851x over previous
import functools

import jax
import jax.numpy as jnp
from jax import lax
from jax.experimental import pallas as pl
from jax.experimental.pallas import tpu as pltpu
from jax.experimental.pallas import tpu_sc as plsc

N = 10000
D = 128
H = 64
NSUB = 32
RPS = 313
NPAD = NSUB * RPS
K = 64
KBUF = K + 16
ZROW = N

_MESH = plsc.VectorSubcoreMesh(core_axis_name="c", subcore_axis_name="s")


def _worker_id():
    return lax.axis_index("s") * 2 + lax.axis_index("c")


_SC_PARAMS = pltpu.CompilerParams(
    needs_layout_passes=False, use_tc_tiling_on_sc=False)


def _extract_body(a_hbm, idx_hbm, rowbuf, idxbuf):
    base = _worker_id() * RPS
    fill = jnp.full((16,), ZROW, jnp.int32)
    lanes = lax.iota(jnp.int32, 16)

    def row_loop(r, carry):
        rr = base + r
        rr_src = jnp.minimum(rr, N - 1)
        pltpu.sync_copy(a_hbm.at[pl.ds(rr_src * N, N)], rowbuf)
        for q in range(KBUF // 16):
            idxbuf[pl.ds(q * 16, 16)] = fill

        def chunk(c, ptr):
            v = rowbuf[pl.ds(c * 16, 16)]
            m = v != 0.0
            ranks = plsc.cumsum(jnp.where(m, 1, 0))
            plsc.store_scatter(idxbuf, [ptr + ranks - 1], lanes + c * 16,
                               mask=m)
            return jnp.minimum(ptr + jnp.max(ranks), K)

        lax.fori_loop(0, N // 16, chunk, 0)
        pltpu.sync_copy(idxbuf.at[pl.ds(0, K)], idx_hbm.at[pl.ds(rr * K, K)])
        return carry

    lax.fori_loop(0, RPS, row_loop, 0)


_extract = functools.partial(
    pl.kernel,
    out_type=jax.ShapeDtypeStruct((NPAD * K,), jnp.int32),
    mesh=_MESH,
    compiler_params=_SC_PARAMS,
    scratch_types=[
        pltpu.VMEM((N,), jnp.float32),
        pltpu.VMEM((KBUF,), jnp.int32),
    ],
)(_extract_body)


def _gmax_body(pool_hbm, idx_hbm, neigh_hbm, idxv, rows, accbuf, sem):
    base = _worker_id() * RPS

    def row_loop(r, carry):
        rr = base + r
        pltpu.sync_copy(idx_hbm.at[pl.ds(rr * K, K)], idxv)
        pltpu.async_copy(pool_hbm.at[idxv], rows, sem).wait()
        acc = [jnp.zeros((16,), jnp.float32) for _ in range(4)]
        for j in range(K):
            for q in range(4):
                acc[q] = jnp.maximum(acc[q], rows[j, pl.ds(q * 16, 16)])
        for q in range(4):
            accbuf[pl.ds(q * 16, 16)] = acc[q]
        pltpu.sync_copy(accbuf, neigh_hbm.at[pl.ds(rr * H, H)])
        return carry

    lax.fori_loop(0, RPS, row_loop, 0)


_gmax = functools.partial(
    pl.kernel,
    out_type=jax.ShapeDtypeStruct((NPAD * H,), jnp.float32),
    mesh=_MESH,
    compiler_params=_SC_PARAMS,
    scratch_types=[
        pltpu.VMEM((K,), jnp.int32),
        pltpu.VMEM((K, H), jnp.float32),
        pltpu.VMEM((H,), jnp.float32),
        pltpu.SemaphoreType.DMA,
    ],
)(_gmax_body)


_BLK = 1000


def _pre_tc(x_ref, wp_ref, bp_ref, pool_ref):
    p = jnp.dot(x_ref[...], wp_ref[...], preferred_element_type=jnp.float32)
    pool_ref[...] = jnp.maximum(p + bp_ref[...], 0.0)


def _post_tc(h_ref, ng_ref, ws_ref, bs_ref, wn_ref, bn_ref, out_ref):
    s = jnp.dot(h_ref[...], ws_ref[...], preferred_element_type=jnp.float32)
    s = s + bs_ref[...]
    nn = jnp.dot(ng_ref[...], wn_ref[...], preferred_element_type=jnp.float32)
    nn = nn + bn_ref[...]
    o = jnp.maximum(jnp.concatenate([s, nn], axis=1), 0.0)
    nrm = jnp.sqrt(jnp.sum(o * o, axis=1, keepdims=True))
    out_ref[...] = o / jnp.maximum(nrm, 1e-12)


def _row_spec(width):
    return pl.BlockSpec((_BLK, width), lambda i: (i, 0))


def _full_spec(shape):
    return pl.BlockSpec(shape, lambda i: (0, 0))


def _pre(h, wp, bp):
    cin = h.shape[1]
    return pl.pallas_call(
        _pre_tc,
        grid=(N // _BLK,),
        in_specs=[_row_spec(cin), _full_spec((cin, H)), _full_spec((1, H))],
        out_specs=_row_spec(H),
        out_shape=jax.ShapeDtypeStruct((N, H), jnp.float32),
    )(h, wp, bp.reshape(1, H))


def _post(h, neigh, ws, bs, wn, bn):
    cin = h.shape[1]
    return pl.pallas_call(
        _post_tc,
        grid=(N // _BLK,),
        in_specs=[
            _row_spec(cin), _row_spec(H),
            _full_spec((cin, H)), _full_spec((1, H)),
            _full_spec((H, H)), _full_spec((1, H)),
        ],
        out_specs=_row_spec(2 * H),
        out_shape=jax.ShapeDtypeStruct((N, 2 * H), jnp.float32),
    )(h, neigh, ws, bs.reshape(1, H), wn, bn.reshape(1, H))


def _pad_pool(pool):
    return jnp.concatenate(
        [pool, jnp.zeros((NPAD - N, H), jnp.float32)], axis=0)


def kernel(x, A, W_pool0, b_pool0, W_self0, b_self0, W_neigh0, b_neigh0,
           W_pool1, b_pool1, W_self1, b_self1, W_neigh1, b_neigh1):
    idx = _extract(A.reshape(-1))

    pool0 = _pre(x, W_pool0, b_pool0)
    neigh0 = _gmax(_pad_pool(pool0), idx).reshape(NPAD, H)[:N]
    h1 = _post(x, neigh0, W_self0, b_self0, W_neigh0, b_neigh0)

    pool1 = _pre(h1, W_pool1, b_pool1)
    neigh1 = _gmax(_pad_pool(pool1), idx).reshape(NPAD, H)[:N]
    return _post(h1, neigh1, W_self1, b_self1, W_neigh1, b_neigh1)


# speedup vs baseline: 9.4868x; 1.0442x over previous
import functools

import jax
import jax.numpy as jnp
from jax import lax
from jax.experimental import pallas as pl
from jax.experimental.pallas import tpu as pltpu
from jax.experimental.pallas import tpu_sc as plsc

N = 10000
D = 128
H = 64
NSUB = 32
RPS = 320
NPAD = NSUB * RPS
K = 64
KBUF = K + 16
ZROW = N
GC = 5
RING = 4
NPAIR = RPS // 2

_MESH = plsc.VectorSubcoreMesh(core_axis_name="c", subcore_axis_name="s")
_SC_PARAMS = pltpu.CompilerParams(
    needs_layout_passes=False, use_tc_tiling_on_sc=False)


def _worker_id():
    return lax.axis_index("s") * 2 + lax.axis_index("c")


def _extract_body(a_hbm, idx_hbm, rowbuf, idxbuf, sem0, sem1):
    base = _worker_id() * RPS
    fill = jnp.full((16,), ZROW, jnp.int32)
    lanes = lax.iota(jnp.int32, 16)
    sems = (sem0, sem1)

    def row_dma(r, slot):
        src = jnp.minimum(base + r, N - 1) * N
        return pltpu.make_async_copy(
            a_hbm.at[pl.ds(src, N)], rowbuf.at[slot], sems[slot])

    row_dma(0, 0).start()

    def scan_row(slot, r):
        rr = base + r
        for q in range(KBUF // 16):
            idxbuf[pl.ds(q * 16, 16)] = fill

        @pl.when(rr < N)
        def _():
            def group(gidx, ptr):
                off = gidx * (GC * 16)
                ms = [rowbuf[slot, pl.ds(off + k * 16, 16)] != 0.0
                      for k in range(GC)]
                mo = ms[0]
                for k in range(1, GC):
                    mo = jnp.logical_or(mo, ms[k])

                def hit(p):
                    for k in range(GC):
                        def chit(pp, mk=ms[k], ko=off + k * 16):
                            ranks = plsc.cumsum(jnp.where(mk, 1, 0))
                            plsc.store_scatter(
                                idxbuf, [pp + ranks - 1], lanes + ko, mask=mk)
                            return jnp.minimum(pp + jnp.max(ranks), K)
                        p = lax.cond(jnp.any(ms[k]), chit, lambda pp: pp, p)
                    return p

                return lax.cond(jnp.any(mo), hit, lambda p: p, ptr)

            lax.fori_loop(0, (N // 16) // GC, group, 0)

        pltpu.sync_copy(idxbuf.at[pl.ds(0, K)], idx_hbm.at[pl.ds(rr * K, K)])

    def row_pair(r2, carry):
        for slot in range(2):
            r = r2 * 2 + slot
            row_dma(r, slot).wait()
            nxt = r + 1

            @pl.when(nxt < RPS)
            def _(nslot=(slot + 1) % 2, nxt=nxt):
                row_dma(nxt, nslot).start()

            scan_row(slot, r)
        return carry

    lax.fori_loop(0, RPS // 2, row_pair, 0)


_extract = functools.partial(
    pl.kernel,
    out_type=jax.ShapeDtypeStruct((NPAD * K,), jnp.int32),
    mesh=_MESH,
    compiler_params=_SC_PARAMS,
    scratch_types=[
        pltpu.VMEM((2, N), jnp.float32),
        pltpu.VMEM((KBUF,), jnp.int32),
        pltpu.SemaphoreType.DMA,
        pltpu.SemaphoreType.DMA,
    ],
)(_extract_body)


def _gmax_body(pool_hbm, idx_hbm, neigh_hbm, idxv, rows, outv,
               gs0, gs1, gs2, gs3):
    base = _worker_id() * RPS
    gsems = (gs0, gs1, gs2, gs3)
    pltpu.sync_copy(idx_hbm.at[pl.ds(base * K, RPS * K)], idxv)

    def pair_dma(p, slot):
        return pltpu.make_async_copy(
            pool_hbm.at[idxv.at[pl.ds(p * 2 * K, 2 * K)]],
            rows.at[slot], gsems[slot])

    for pre in range(RING - 1):
        pair_dma(pre, pre).start()

    def ring_cycle(g, carry):
        for slot in range(RING):
            p = g * RING + slot
            pair_dma(p, slot).wait()
            nxt = p + RING - 1

            @pl.when(nxt < NPAIR)
            def _(nslot=(slot + RING - 1) % RING, nxt=nxt):
                pair_dma(nxt, nslot).start()

            for half in range(2):
                acc = [jnp.zeros((16,), jnp.float32) for _ in range(4)]
                for j in range(K):
                    for q in range(4):
                        acc[q] = jnp.maximum(
                            acc[q], rows[slot, half * K + j, pl.ds(q * 16, 16)])
                for q in range(4):
                    outv[pl.ds((p * 2 + half) * H + q * 16, 16)] = acc[q]
        return carry

    lax.fori_loop(0, NPAIR // RING, ring_cycle, 0)
    pltpu.sync_copy(outv, neigh_hbm.at[pl.ds(base * H, RPS * H)])


_gmax = functools.partial(
    pl.kernel,
    out_type=jax.ShapeDtypeStruct((NPAD * H,), jnp.float32),
    mesh=_MESH,
    compiler_params=_SC_PARAMS,
    scratch_types=[
        pltpu.VMEM((RPS * K,), jnp.int32),
        pltpu.VMEM((RING, 2 * K, H), jnp.float32),
        pltpu.VMEM((RPS * H,), jnp.float32),
        pltpu.SemaphoreType.DMA,
        pltpu.SemaphoreType.DMA,
        pltpu.SemaphoreType.DMA,
        pltpu.SemaphoreType.DMA,
    ],
)(_gmax_body)


_BLK = 1000


def _pre_tc(x_ref, wp_ref, bp_ref, pool_ref):
    p = jnp.dot(x_ref[...], wp_ref[...], preferred_element_type=jnp.float32)
    pool_ref[...] = jnp.maximum(p + bp_ref[...], 0.0)


def _post_tc(h_ref, ng_ref, ws_ref, bs_ref, wn_ref, bn_ref, out_ref):
    s = jnp.dot(h_ref[...], ws_ref[...], preferred_element_type=jnp.float32)
    s = s + bs_ref[...]
    nn = jnp.dot(ng_ref[...], wn_ref[...], preferred_element_type=jnp.float32)
    nn = nn + bn_ref[...]
    o = jnp.maximum(jnp.concatenate([s, nn], axis=1), 0.0)
    nrm = jnp.sqrt(jnp.sum(o * o, axis=1, keepdims=True))
    out_ref[...] = o / jnp.maximum(nrm, 1e-12)


def _row_spec(width):
    return pl.BlockSpec((_BLK, width), lambda i: (i, 0))


def _full_spec(shape):
    return pl.BlockSpec(shape, lambda i: (0, 0))


def _pre(h, wp, bp):
    cin = h.shape[1]
    return pl.pallas_call(
        _pre_tc,
        grid=(N // _BLK,),
        in_specs=[_row_spec(cin), _full_spec((cin, H)), _full_spec((1, H))],
        out_specs=_row_spec(H),
        out_shape=jax.ShapeDtypeStruct((N, H), jnp.float32),
    )(h, wp, bp.reshape(1, H))


def _post(h, neigh, ws, bs, wn, bn):
    cin = h.shape[1]
    return pl.pallas_call(
        _post_tc,
        grid=(N // _BLK,),
        in_specs=[
            _row_spec(cin), _row_spec(H),
            _full_spec((cin, H)), _full_spec((1, H)),
            _full_spec((H, H)), _full_spec((1, H)),
        ],
        out_specs=_row_spec(2 * H),
        out_shape=jax.ShapeDtypeStruct((N, 2 * H), jnp.float32),
    )(h, neigh, ws, bs.reshape(1, H), wn, bn.reshape(1, H))


def _pad_pool(pool):
    return jnp.concatenate(
        [pool, jnp.zeros((NPAD - N, H), jnp.float32)], axis=0)


def kernel(x, A, W_pool0, b_pool0, W_self0, b_self0, W_neigh0, b_neigh0,
           W_pool1, b_pool1, W_self1, b_self1, W_neigh1, b_neigh1):
    idx = _extract(A.reshape(-1))

    pool0 = _pre(x, W_pool0, b_pool0)
    neigh0 = _gmax(_pad_pool(pool0), idx).reshape(NPAD, H)[:N]
    h1 = _post(x, neigh0, W_self0, b_self0, W_neigh0, b_neigh0)

    pool1 = _pre(h1, W_pool1, b_pool1)
    neigh1 = _gmax(_pad_pool(pool1), idx).reshape(NPAD, H)[:N]
    return _post(h1, neigh1, W_self1, b_self1, W_neigh1, b_neigh1)


# speedup vs baseline: 81.0111x; 8.5394x over previous
import functools

import jax
import jax.numpy as jnp
from jax import lax
from jax.experimental import pallas as pl
from jax.experimental.pallas import tpu as pltpu
from jax.experimental.pallas import tpu_sc as plsc

N = 10000
D = 128
H = 64
NSUB = 32
RPS = 320
NPAD = NSUB * RPS
K = 64
KBUF = K + 16
ZROW = N
GC = 5
RING = 4
NPAIR = RPS // 2

_MESH = plsc.VectorSubcoreMesh(core_axis_name="c", subcore_axis_name="s")
_SC_PARAMS = pltpu.CompilerParams(
    needs_layout_passes=False, use_tc_tiling_on_sc=False)


def _worker_id():
    return lax.axis_index("s") * 2 + lax.axis_index("c")


def _extract_body(a_hbm, idx_hbm, rowbuf, idxbuf, sem0, sem1):
    base = _worker_id() * RPS
    fill = jnp.full((16,), ZROW, jnp.int32)
    lanes = lax.iota(jnp.int32, 16)
    sems = (sem0, sem1)

    def row_dma(r, slot):
        src = jnp.minimum(base + r, N - 1) * N
        return pltpu.make_async_copy(
            a_hbm.at[pl.ds(src, N)], rowbuf.at[slot], sems[slot])

    row_dma(0, 0).start()

    def scan_row(slot, r):
        rr = base + r
        for q in range(KBUF // 16):
            idxbuf[pl.ds(q * 16, 16)] = fill

        @pl.when(rr < N)
        def _():
            def group(gidx, ptr):
                off = gidx * (GC * 16)
                ms = [rowbuf[slot, pl.ds(off + k * 16, 16)] != 0.0
                      for k in range(GC)]
                mo = ms[0]
                for k in range(1, GC):
                    mo = jnp.logical_or(mo, ms[k])

                def hit(p):
                    for k in range(GC):
                        def chit(pp, mk=ms[k], ko=off + k * 16):
                            ranks = plsc.cumsum(jnp.where(mk, 1, 0))
                            plsc.store_scatter(
                                idxbuf, [pp + ranks - 1], lanes + ko, mask=mk)
                            return jnp.minimum(pp + jnp.max(ranks), K)
                        p = lax.cond(jnp.any(ms[k]), chit, lambda pp: pp, p)
                    return p

                return lax.cond(jnp.any(mo), hit, lambda p: p, ptr)

            lax.fori_loop(0, (N // 16) // GC, group, 0)

        pltpu.sync_copy(idxbuf.at[pl.ds(0, K)], idx_hbm.at[pl.ds(rr * K, K)])

    def row_pair(r2, carry):
        for slot in range(2):
            r = r2 * 2 + slot
            row_dma(r, slot).wait()
            nxt = r + 1

            @pl.when(nxt < RPS)
            def _(nslot=(slot + 1) % 2, nxt=nxt):
                row_dma(nxt, nslot).start()

            scan_row(slot, r)
        return carry

    lax.fori_loop(0, RPS // 2, row_pair, 0)


_extract = functools.partial(
    pl.kernel,
    out_type=jax.ShapeDtypeStruct((NPAD * K,), jnp.int32),
    mesh=_MESH,
    compiler_params=_SC_PARAMS,
    scratch_types=[
        pltpu.VMEM((2, N), jnp.float32),
        pltpu.VMEM((KBUF,), jnp.int32),
        pltpu.SemaphoreType.DMA,
        pltpu.SemaphoreType.DMA,
    ],
)(_extract_body)


RG = NPAD // 4
CW = 8
IB = 256


def _permute16(x, idx):
    dn = lax.GatherDimensionNumbers(
        offset_dims=(), collapsed_slice_dims=(0,), start_index_map=(0,))
    return lax.gather(x, idx[:, None], dn, (1,),
                      mode=lax.GatherScatterMode.PROMISE_IN_BOUNDS)


def _gmax_body(pool_hbm, idx_hbm, neigh_hbm, pool8, idxv, out8):
    wid = _worker_id()
    rg = wid // 8
    cg = wid % 8
    row0 = rg * RG
    lanes = lax.iota(jnp.int32, 16)
    cols8 = jnp.where(lanes < 8, lanes, lanes - 8)
    lanes_lo = lanes < 8
    fold_idx = jnp.where(lanes < 8, lanes + 8, lanes)
    odd = jnp.where(lanes < 8, 0, 1)
    pats = [odd + 2 * k for k in range(8)]

    pltpu.sync_copy(pool_hbm.at[pl.ds(0, NPAD), pl.ds(cg * CW, CW)], pool8)

    def batch_loop(b, carry):
        boff = b * IB
        pltpu.sync_copy(
            idx_hbm.at[pl.ds((row0 + boff) * K, IB * K)], idxv)

        def row_loop(rl, carry2):
            r = rl * 2
            accs = [jnp.full((16,), 0.0, jnp.float32) for _ in range(2)]
            for half in range(2):
                for ch in range(K // 16):
                    slots = idxv[pl.ds((r + half) * K + ch * 16, 16)]
                    for k in range(8):
                        rowsvec = _permute16(slots, pats[k])
                        g = plsc.load_gather(pool8, [rowsvec, cols8])
                        accs[half] = jnp.maximum(accs[half], g)
                a = accs[half]
                fold = jnp.maximum(a, _permute16(a, fold_idx))
                rowspl = lanes * 0 + (boff + r + half)
                plsc.store_scatter(out8, [rowspl, cols8], fold,
                                   mask=lanes_lo)
            return carry2

        lax.fori_loop(0, IB // 2, row_loop, 0)
        return carry

    lax.fori_loop(0, RG // IB, batch_loop, 0)
    pltpu.sync_copy(
        out8, neigh_hbm.at[pl.ds(row0, RG), pl.ds(cg * CW, CW)])


_gmax = functools.partial(
    pl.kernel,
    out_type=jax.ShapeDtypeStruct((NPAD, H), jnp.float32),
    mesh=_MESH,
    compiler_params=_SC_PARAMS,
    scratch_types=[
        pltpu.VMEM((NPAD, CW), jnp.float32),
        pltpu.VMEM((IB * K,), jnp.int32),
        pltpu.VMEM((RG, CW), jnp.float32),
    ],
)(_gmax_body)


_BLK = 1000


def _pre_tc(x_ref, wp_ref, bp_ref, pool_ref):
    p = jnp.dot(x_ref[...], wp_ref[...], preferred_element_type=jnp.float32)
    pool_ref[...] = jnp.maximum(p + bp_ref[...], 0.0)


def _post_tc(h_ref, ng_ref, ws_ref, bs_ref, wn_ref, bn_ref, out_ref):
    s = jnp.dot(h_ref[...], ws_ref[...], preferred_element_type=jnp.float32)
    s = s + bs_ref[...]
    nn = jnp.dot(ng_ref[...], wn_ref[...], preferred_element_type=jnp.float32)
    nn = nn + bn_ref[...]
    o = jnp.maximum(jnp.concatenate([s, nn], axis=1), 0.0)
    nrm = jnp.sqrt(jnp.sum(o * o, axis=1, keepdims=True))
    out_ref[...] = o / jnp.maximum(nrm, 1e-12)


def _row_spec(width):
    return pl.BlockSpec((_BLK, width), lambda i: (i, 0))


def _full_spec(shape):
    return pl.BlockSpec(shape, lambda i: (0, 0))


def _pre(h, wp, bp):
    cin = h.shape[1]
    return pl.pallas_call(
        _pre_tc,
        grid=(N // _BLK,),
        in_specs=[_row_spec(cin), _full_spec((cin, H)), _full_spec((1, H))],
        out_specs=_row_spec(H),
        out_shape=jax.ShapeDtypeStruct((N, H), jnp.float32),
    )(h, wp, bp.reshape(1, H))


def _post(h, neigh, ws, bs, wn, bn):
    cin = h.shape[1]
    return pl.pallas_call(
        _post_tc,
        grid=(N // _BLK,),
        in_specs=[
            _row_spec(cin), _row_spec(H),
            _full_spec((cin, H)), _full_spec((1, H)),
            _full_spec((H, H)), _full_spec((1, H)),
        ],
        out_specs=_row_spec(2 * H),
        out_shape=jax.ShapeDtypeStruct((N, 2 * H), jnp.float32),
    )(h, neigh, ws, bs.reshape(1, H), wn, bn.reshape(1, H))


def _pad_pool(pool):
    return jnp.concatenate(
        [pool, jnp.zeros((NPAD - N, H), jnp.float32)], axis=0)


def kernel(x, A, W_pool0, b_pool0, W_self0, b_self0, W_neigh0, b_neigh0,
           W_pool1, b_pool1, W_self1, b_self1, W_neigh1, b_neigh1):
    idx = _extract(A.reshape(-1))

    pool0 = _pre(x, W_pool0, b_pool0)
    neigh0 = _gmax(_pad_pool(pool0), idx)[:N]
    h1 = _post(x, neigh0, W_self0, b_self0, W_neigh0, b_neigh0)

    pool1 = _pre(h1, W_pool1, b_pool1)
    neigh1 = _gmax(_pad_pool(pool1), idx)[:N]
    return _post(h1, neigh1, W_self1, b_self1, W_neigh1, b_neigh1)


# speedup vs baseline: 140.1474x; 1.7300x over previous
import functools

import jax
import jax.numpy as jnp
from jax import lax
from jax.experimental import pallas as pl
from jax.experimental.pallas import tpu as pltpu
from jax.experimental.pallas import tpu_sc as plsc

N = 10000
D = 128
H = 64
NSUB = 32
RPS = 320
NPAD = NSUB * RPS
K = 64
KBUF = K + 16
ZROW = N
GC = 5
RING = 4
NPAIR = RPS // 2

_MESH = plsc.VectorSubcoreMesh(core_axis_name="c", subcore_axis_name="s")
_SC_PARAMS = pltpu.CompilerParams(
    needs_layout_passes=False, use_tc_tiling_on_sc=False)


def _worker_id():
    return lax.axis_index("s") * 2 + lax.axis_index("c")


def _extract_body(a_hbm, idx_hbm, rowbuf, idxbuf, sem0, sem1):
    base = _worker_id() * RPS
    fill = jnp.full((16,), ZROW, jnp.int32)
    lanes = lax.iota(jnp.int32, 16)
    sems = (sem0, sem1)

    def row_dma(r, slot):
        src = jnp.minimum(base + r, N - 1) * N
        return pltpu.make_async_copy(
            a_hbm.at[pl.ds(src, N)], rowbuf.at[slot], sems[slot])

    row_dma(0, 0).start()

    def scan_row(slot, r):
        rr = base + r
        for q in range(KBUF // 16):
            idxbuf[pl.ds(q * 16, 16)] = fill

        @pl.when(rr < N)
        def _():
            def group(gidx, ptr):
                off = gidx * 400
                vs = [rowbuf[slot, pl.ds(off + c * 16, 16)]
                      for c in range(25)]
                S = vs[0]
                W = vs[0]
                for c in range(1, 25):
                    S = S + vs[c]
                    W = W + vs[c] * jnp.float32(c + 1)

                def chunk_at(c):
                    return rowbuf[slot, pl.ds(off + c * 16, 16)] != 0.0

                def hit(p):
                    def clean(pp):
                        mk = S != 0.0
                        cols = (W - 1.0).astype(jnp.int32) * 16 + (
                            lanes + off)
                        ranks = plsc.cumsum(jnp.where(mk, 1, 0))
                        plsc.store_scatter(idxbuf, [pp + ranks - 1], cols,
                                           mask=mk)
                        return jnp.minimum(pp + jnp.max(ranks), K)

                    def collide(pp):
                        for c in range(25):
                            def chit(qq, c=c):
                                m = chunk_at(c)
                                ranks = plsc.cumsum(jnp.where(m, 1, 0))
                                plsc.store_scatter(
                                    idxbuf, [qq + ranks - 1],
                                    lanes + (off + c * 16), mask=m)
                                return jnp.minimum(qq + jnp.max(ranks), K)
                            pp = lax.cond(jnp.any(chunk_at(c)), chit,
                                          lambda qq: qq, pp)
                        return pp

                    return lax.cond(jnp.any(S > 1.5), collide, clean, p)

                return lax.cond(jnp.any(S != 0.0), hit, lambda p: p, ptr)

            lax.fori_loop(0, 25, group, 0)

        pltpu.sync_copy(idxbuf.at[pl.ds(0, K)], idx_hbm.at[pl.ds(rr * K, K)])

    def row_pair(r2, carry):
        for slot in range(2):
            r = r2 * 2 + slot
            row_dma(r, slot).wait()
            nxt = r + 1

            @pl.when(nxt < RPS)
            def _(nslot=(slot + 1) % 2, nxt=nxt):
                row_dma(nxt, nslot).start()

            scan_row(slot, r)
        return carry

    lax.fori_loop(0, RPS // 2, row_pair, 0)


_extract = functools.partial(
    pl.kernel,
    out_type=jax.ShapeDtypeStruct((NPAD * K,), jnp.int32),
    mesh=_MESH,
    compiler_params=_SC_PARAMS,
    scratch_types=[
        pltpu.VMEM((2, N), jnp.float32),
        pltpu.VMEM((KBUF,), jnp.int32),
        pltpu.SemaphoreType.DMA,
        pltpu.SemaphoreType.DMA,
    ],
)(_extract_body)


RG = NPAD // 4
CW = 8
IB = 256


def _permute16(x, idx):
    dn = lax.GatherDimensionNumbers(
        offset_dims=(), collapsed_slice_dims=(0,), start_index_map=(0,))
    return lax.gather(x, idx[:, None], dn, (1,),
                      mode=lax.GatherScatterMode.PROMISE_IN_BOUNDS)


def _gmax_body(pool_hbm, idx_hbm, neigh_hbm, pool8, idxv, out8):
    wid = _worker_id()
    rg = wid // 8
    cg = wid % 8
    row0 = rg * RG
    lanes = lax.iota(jnp.int32, 16)
    cols8 = jnp.where(lanes < 8, lanes, lanes - 8)
    lanes_lo = lanes < 8
    fold_idx = jnp.where(lanes < 8, lanes + 8, lanes)
    odd = jnp.where(lanes < 8, 0, 1)
    pats = [odd + 2 * k for k in range(8)]

    pltpu.sync_copy(pool_hbm.at[pl.ds(0, NPAD), pl.ds(cg * CW, CW)], pool8)

    def batch_loop(b, carry):
        boff = b * IB
        pltpu.sync_copy(
            idx_hbm.at[pl.ds((row0 + boff) * K, IB * K)], idxv)

        def row_loop(rl, carry2):
            r = rl * 2
            accs = [jnp.full((16,), 0.0, jnp.float32) for _ in range(2)]
            for half in range(2):
                for ch in range(K // 16):
                    slots = idxv[pl.ds((r + half) * K + ch * 16, 16)]
                    for k in range(8):
                        rowsvec = _permute16(slots, pats[k])
                        g = plsc.load_gather(pool8, [rowsvec, cols8])
                        accs[half] = jnp.maximum(accs[half], g)
                a = accs[half]
                fold = jnp.maximum(a, _permute16(a, fold_idx))
                rowspl = lanes * 0 + (boff + r + half)
                plsc.store_scatter(out8, [rowspl, cols8], fold,
                                   mask=lanes_lo)
            return carry2

        lax.fori_loop(0, IB // 2, row_loop, 0)
        return carry

    lax.fori_loop(0, RG // IB, batch_loop, 0)
    pltpu.sync_copy(
        out8, neigh_hbm.at[pl.ds(row0, RG), pl.ds(cg * CW, CW)])


_gmax = functools.partial(
    pl.kernel,
    out_type=jax.ShapeDtypeStruct((NPAD, H), jnp.float32),
    mesh=_MESH,
    compiler_params=_SC_PARAMS,
    scratch_types=[
        pltpu.VMEM((NPAD, CW), jnp.float32),
        pltpu.VMEM((IB * K,), jnp.int32),
        pltpu.VMEM((RG, CW), jnp.float32),
    ],
)(_gmax_body)


_BLK = 1000


def _pre_tc(x_ref, wp_ref, bp_ref, pool_ref):
    p = jnp.dot(x_ref[...], wp_ref[...], preferred_element_type=jnp.float32)
    pool_ref[...] = jnp.maximum(p + bp_ref[...], 0.0)


def _post_tc(h_ref, ng_ref, ws_ref, bs_ref, wn_ref, bn_ref, out_ref):
    s = jnp.dot(h_ref[...], ws_ref[...], preferred_element_type=jnp.float32)
    s = s + bs_ref[...]
    nn = jnp.dot(ng_ref[...], wn_ref[...], preferred_element_type=jnp.float32)
    nn = nn + bn_ref[...]
    o = jnp.maximum(jnp.concatenate([s, nn], axis=1), 0.0)
    nrm = jnp.sqrt(jnp.sum(o * o, axis=1, keepdims=True))
    out_ref[...] = o / jnp.maximum(nrm, 1e-12)


def _row_spec(width):
    return pl.BlockSpec((_BLK, width), lambda i: (i, 0))


def _full_spec(shape):
    return pl.BlockSpec(shape, lambda i: (0, 0))


def _pre(h, wp, bp):
    cin = h.shape[1]
    return pl.pallas_call(
        _pre_tc,
        grid=(N // _BLK,),
        in_specs=[_row_spec(cin), _full_spec((cin, H)), _full_spec((1, H))],
        out_specs=_row_spec(H),
        out_shape=jax.ShapeDtypeStruct((N, H), jnp.float32),
    )(h, wp, bp.reshape(1, H))


def _post(h, neigh, ws, bs, wn, bn):
    cin = h.shape[1]
    return pl.pallas_call(
        _post_tc,
        grid=(N // _BLK,),
        in_specs=[
            _row_spec(cin), _row_spec(H),
            _full_spec((cin, H)), _full_spec((1, H)),
            _full_spec((H, H)), _full_spec((1, H)),
        ],
        out_specs=_row_spec(2 * H),
        out_shape=jax.ShapeDtypeStruct((N, 2 * H), jnp.float32),
    )(h, neigh, ws, bs.reshape(1, H), wn, bn.reshape(1, H))


def _pad_pool(pool):
    return jnp.concatenate(
        [pool, jnp.zeros((NPAD - N, H), jnp.float32)], axis=0)


def kernel(x, A, W_pool0, b_pool0, W_self0, b_self0, W_neigh0, b_neigh0,
           W_pool1, b_pool1, W_self1, b_self1, W_neigh1, b_neigh1):
    idx = _extract(A.reshape(-1))

    pool0 = _pre(x, W_pool0, b_pool0)
    neigh0 = _gmax(_pad_pool(pool0), idx)[:N]
    h1 = _post(x, neigh0, W_self0, b_self0, W_neigh0, b_neigh0)

    pool1 = _pre(h1, W_pool1, b_pool1)
    neigh1 = _gmax(_pad_pool(pool1), idx)[:N]
    return _post(h1, neigh1, W_self1, b_self1, W_neigh1, b_neigh1)


# speedup vs baseline: 140.1718x; 1.0002x over previous
import functools

import jax
import jax.numpy as jnp
from jax import lax
from jax.experimental import pallas as pl
from jax.experimental.pallas import tpu as pltpu
from jax.experimental.pallas import tpu_sc as plsc

N = 10000
D = 128
H = 64
NSUB = 32
RPS = 320
NPAD = NSUB * RPS
K = 64
KBUF = K + 16
ZROW = N

_MESH = plsc.VectorSubcoreMesh(core_axis_name="c", subcore_axis_name="s")
_SC_PARAMS = pltpu.CompilerParams(
    needs_layout_passes=False, use_tc_tiling_on_sc=False)


def _worker_id():
    return lax.axis_index("s") * 2 + lax.axis_index("c")


def _extract_body(a_hbm, idx_hbm, rowbuf, idxbuf, sem0, sem1):
    base = _worker_id() * RPS
    fill = jnp.full((16,), ZROW, jnp.int32)
    lanes = lax.iota(jnp.int32, 16)
    sems = (sem0, sem1)

    def row_dma(r, slot):
        src = jnp.minimum(base + r, N - 1) * N
        return pltpu.make_async_copy(
            a_hbm.at[pl.ds(src, N)], rowbuf.at[slot], sems[slot])

    row_dma(0, 0).start()

    def scan_row(slot, r):
        rr = base + r
        for q in range(KBUF // 16):
            idxbuf[pl.ds(q * 16, 16)] = fill

        @pl.when(rr < N)
        def _():
            def group(gidx, ptr):
                off = gidx * 400
                vs = [rowbuf[slot, pl.ds(off + c * 16, 16)]
                      for c in range(25)]
                S = vs[0]
                W = vs[0]
                for c in range(1, 25):
                    S = S + vs[c]
                    W = W + vs[c] * jnp.float32(c + 1)

                def chunk_at(c):
                    return rowbuf[slot, pl.ds(off + c * 16, 16)] != 0.0

                def hit(p):
                    def clean(pp):
                        mk = S != 0.0
                        cols = (W - 1.0).astype(jnp.int32) * 16 + (
                            lanes + off)
                        ranks = plsc.cumsum(jnp.where(mk, 1, 0))
                        plsc.store_scatter(idxbuf, [pp + ranks - 1], cols,
                                           mask=mk)
                        return jnp.minimum(pp + jnp.max(ranks), K)

                    def collide(pp):
                        for c in range(25):
                            def chit(qq, c=c):
                                m = chunk_at(c)
                                ranks = plsc.cumsum(jnp.where(m, 1, 0))
                                plsc.store_scatter(
                                    idxbuf, [qq + ranks - 1],
                                    lanes + (off + c * 16), mask=m)
                                return jnp.minimum(qq + jnp.max(ranks), K)
                            pp = lax.cond(jnp.any(chunk_at(c)), chit,
                                          lambda qq: qq, pp)
                        return pp

                    return lax.cond(jnp.any(S > 1.5), collide, clean, p)

                return lax.cond(jnp.any(S != 0.0), hit, lambda p: p, ptr)

            lax.fori_loop(0, 25, group, 0)

        pltpu.sync_copy(idxbuf.at[pl.ds(0, K)], idx_hbm.at[pl.ds(rr * K, K)])

    def row_pair(r2, carry):
        for slot in range(2):
            r = r2 * 2 + slot
            row_dma(r, slot).wait()
            nxt = r + 1

            @pl.when(nxt < RPS)
            def _(nslot=(slot + 1) % 2, nxt=nxt):
                row_dma(nxt, nslot).start()

            scan_row(slot, r)
        return carry

    lax.fori_loop(0, RPS // 2, row_pair, 0)


_extract = functools.partial(
    pl.kernel,
    out_type=jax.ShapeDtypeStruct((NPAD * K,), jnp.int32),
    mesh=_MESH,
    compiler_params=_SC_PARAMS,
    scratch_types=[
        pltpu.VMEM((2, N), jnp.float32),
        pltpu.VMEM((KBUF,), jnp.int32),
        pltpu.SemaphoreType.DMA,
        pltpu.SemaphoreType.DMA,
    ],
)(_extract_body)


RG = NPAD // 4
CW = 8
IB = 256


def _permute16(x, idx):
    dn = lax.GatherDimensionNumbers(
        offset_dims=(), collapsed_slice_dims=(0,), start_index_map=(0,))
    return lax.gather(x, idx[:, None], dn, (1,),
                      mode=lax.GatherScatterMode.PROMISE_IN_BOUNDS)


def _gmax_body(pool_hbm, idx_hbm, neigh_hbm, pool8, idxv, out8):
    wid = _worker_id()
    rg = wid // 8
    cg = wid % 8
    row0 = rg * RG
    lanes = lax.iota(jnp.int32, 16)
    cols8 = jnp.where(lanes < 8, lanes, lanes - 8)
    lanes_lo = lanes < 8
    fold_idx = jnp.where(lanes < 8, lanes + 8, lanes)
    odd = jnp.where(lanes < 8, 0, 1)
    pats = [odd + 2 * k for k in range(8)]

    pltpu.sync_copy(pool_hbm.at[pl.ds(0, NPAD), pl.ds(cg * CW, CW)], pool8)

    def batch_loop(b, carry):
        boff = b * IB
        pltpu.sync_copy(
            idx_hbm.at[pl.ds((row0 + boff) * K, IB * K)], idxv)

        def row_loop(rl, carry2):
            r = rl * 2
            accs = [jnp.full((16,), 0.0, jnp.float32) for _ in range(2)]
            for half in range(2):
                for ch in range(K // 16):
                    slots = idxv[pl.ds((r + half) * K + ch * 16, 16)]
                    for k in range(8):
                        rowsvec = _permute16(slots, pats[k])
                        g = plsc.load_gather(pool8, [rowsvec, cols8])
                        accs[half] = jnp.maximum(accs[half], g)
                a = accs[half]
                fold = jnp.maximum(a, _permute16(a, fold_idx))
                rowspl = lanes * 0 + (boff + r + half)
                plsc.store_scatter(out8, [rowspl, cols8], fold,
                                   mask=lanes_lo)
            return carry2

        lax.fori_loop(0, IB // 2, row_loop, 0)
        return carry

    lax.fori_loop(0, RG // IB, batch_loop, 0)
    pltpu.sync_copy(
        out8, neigh_hbm.at[pl.ds(row0, RG), pl.ds(cg * CW, CW)])


_gmax = functools.partial(
    pl.kernel,
    out_type=jax.ShapeDtypeStruct((NPAD, H), jnp.float32),
    mesh=_MESH,
    compiler_params=_SC_PARAMS,
    scratch_types=[
        pltpu.VMEM((NPAD, CW), jnp.float32),
        pltpu.VMEM((IB * K,), jnp.int32),
        pltpu.VMEM((RG, CW), jnp.float32),
    ],
)(_gmax_body)


_BLK = 1000


def _pre_tc(x_ref, wp_ref, bp_ref, pool_ref):
    p = jnp.dot(x_ref[...], wp_ref[...], preferred_element_type=jnp.float32)
    pool_ref[...] = jnp.maximum(p + bp_ref[...], 0.0)


def _post_tc(h_ref, ng_ref, ws_ref, bs_ref, wn_ref, bn_ref, out_ref):
    s = jnp.dot(h_ref[...], ws_ref[...], preferred_element_type=jnp.float32)
    s = s + bs_ref[...]
    nn = jnp.dot(ng_ref[...], wn_ref[...], preferred_element_type=jnp.float32)
    nn = nn + bn_ref[...]
    o = jnp.maximum(jnp.concatenate([s, nn], axis=1), 0.0)
    nrm = jnp.sqrt(jnp.sum(o * o, axis=1, keepdims=True))
    out_ref[...] = o / jnp.maximum(nrm, 1e-12)


def _row_spec(width):
    return pl.BlockSpec((_BLK, width), lambda i: (i, 0))


def _full_spec(shape):
    return pl.BlockSpec(shape, lambda i: (0, 0))


def _pre(h, wp, bp):
    cin = h.shape[1]
    return pl.pallas_call(
        _pre_tc,
        grid=(N // _BLK,),
        in_specs=[_row_spec(cin), _full_spec((cin, H)), _full_spec((1, H))],
        out_specs=_row_spec(H),
        out_shape=jax.ShapeDtypeStruct((N, H), jnp.float32),
    )(h, wp, bp.reshape(1, H))


def _post(h, neigh, ws, bs, wn, bn):
    cin = h.shape[1]
    return pl.pallas_call(
        _post_tc,
        grid=(N // _BLK,),
        in_specs=[
            _row_spec(cin), _row_spec(H),
            _full_spec((cin, H)), _full_spec((1, H)),
            _full_spec((H, H)), _full_spec((1, H)),
        ],
        out_specs=_row_spec(2 * H),
        out_shape=jax.ShapeDtypeStruct((N, 2 * H), jnp.float32),
    )(h, neigh, ws, bs.reshape(1, H), wn, bn.reshape(1, H))


def _pad_pool(pool):
    return jnp.concatenate(
        [pool, jnp.zeros((NPAD - N, H), jnp.float32)], axis=0)


def kernel(x, A, W_pool0, b_pool0, W_self0, b_self0, W_neigh0, b_neigh0,
           W_pool1, b_pool1, W_self1, b_self1, W_neigh1, b_neigh1):
    idx = _extract(A.reshape(-1))

    pool0 = _pre(x, W_pool0, b_pool0)
    neigh0 = _gmax(_pad_pool(pool0), idx)[:N]
    h1 = _post(x, neigh0, W_self0, b_self0, W_neigh0, b_neigh0)

    pool1 = _pre(h1, W_pool1, b_pool1)
    neigh1 = _gmax(_pad_pool(pool1), idx)[:N]
    return _post(h1, neigh1, W_self1, b_self1, W_neigh1, b_neigh1)
